# Initial kernel scaffold; baseline (speedup 1.0000x reference)
import jax
import jax.numpy as jnp
from jax.experimental import pallas as pl


def kernel(x, edge_index, W0, b0, W1, b1):
    raise NotImplementedError("write your pallas kernel here")


# speedup vs baseline: 4.1254x; 4.1254x over previous
import functools

import jax
import jax.numpy as jnp
from jax import lax
from jax.experimental import pallas as pl
from jax.experimental.pallas import tpu as pltpu
from jax.experimental.pallas import tpu_sc as plsc

N = 10000
E = 320000
D = 128
C = 16

NC = 2
NS = 16
NW = NC * NS
NP = 10240
RPW = NP // NS
EP = NP * NW
EPW = EP // NW
CHUNK = 128
NCH = EPW // CHUNK

_mesh = plsc.VectorSubcoreMesh(core_axis_name="c", subcore_axis_name="s")


@functools.partial(
    pl.kernel,
    out_type=(
        jax.ShapeDtypeStruct((NC, NP, D), jnp.float32),
        jax.ShapeDtypeStruct((NC, NP), jnp.float32),
    ),
    mesh=_mesh,
    scratch_types=[
        pltpu.VMEM((CHUNK,), jnp.int32),
        pltpu.VMEM((CHUNK,), jnp.int32),
        pltpu.VMEM((CHUNK, D), jnp.float32),
        pltpu.VMEM((CHUNK,), jnp.float32),
        pltpu.VMEM_SHARED((NP, D), jnp.float32),
        pltpu.VMEM_SHARED((NP,), jnp.float32),
        pltpu.SemaphoreType.DMA,
    ],
)
def _agg_wide(x_hbm, src_hbm, dst_hbm, zrows_hbm, zvec_hbm, ones_hbm,
              acc_out, deg_out, sidx, didx, rows, ones_v, acc_sh, deg_sh,
              sem):
    c = lax.axis_index("c")
    s = lax.axis_index("s")
    r0 = s * RPW
    pltpu.sync_copy(zrows_hbm, acc_sh.at[pl.ds(r0, RPW)])
    pltpu.sync_copy(zvec_hbm, deg_sh.at[pl.ds(r0, RPW)])
    pltpu.sync_copy(ones_hbm, ones_v)
    plsc.subcore_barrier()

    base = (c * NS + s) * EPW

    def step(t, carry):
        off = base + t * CHUNK
        pltpu.sync_copy(src_hbm.at[pl.ds(off, CHUNK)], sidx)
        pltpu.sync_copy(dst_hbm.at[pl.ds(off, CHUNK)], didx)
        pltpu.async_copy(x_hbm.at[sidx], rows, sem).wait()
        pltpu.sync_copy(rows, acc_sh.at[didx], add=True)
        pltpu.sync_copy(ones_v, deg_sh.at[didx], add=True)
        return carry

    lax.fori_loop(0, NCH, step, 0)
    plsc.subcore_barrier()
    pltpu.sync_copy(acc_sh.at[pl.ds(r0, RPW)], acc_out.at[c, pl.ds(r0, RPW)])
    pltpu.sync_copy(deg_sh.at[pl.ds(r0, RPW)], deg_out.at[c, pl.ds(r0, RPW)])


@functools.partial(
    pl.kernel,
    out_type=jax.ShapeDtypeStruct((NC, NP, C), jnp.float32),
    mesh=_mesh,
    scratch_types=[
        pltpu.VMEM((CHUNK,), jnp.int32),
        pltpu.VMEM((CHUNK,), jnp.int32),
        pltpu.VMEM((CHUNK, C), jnp.float32),
        pltpu.VMEM_SHARED((NP, C), jnp.float32),
        pltpu.VMEM_SHARED((NP, C), jnp.float32),
        pltpu.SemaphoreType.DMA,
    ],
)
def _agg_narrow(g_hbm, src_hbm, dst_hbm, zrows_hbm, acc_out,
                sidx, didx, rows, acc_sh, g_sh, sem):
    c = lax.axis_index("c")
    s = lax.axis_index("s")
    r0 = s * RPW
    pltpu.sync_copy(zrows_hbm, acc_sh.at[pl.ds(r0, RPW)])
    pltpu.sync_copy(g_hbm.at[pl.ds(r0, RPW)], g_sh.at[pl.ds(r0, RPW)])
    plsc.subcore_barrier()

    base = (c * NS + s) * EPW

    def step(t, carry):
        off = base + t * CHUNK
        pltpu.sync_copy(src_hbm.at[pl.ds(off, CHUNK)], sidx)
        pltpu.sync_copy(dst_hbm.at[pl.ds(off, CHUNK)], didx)
        pltpu.async_copy(g_sh.at[sidx], rows, sem).wait()
        pltpu.sync_copy(rows, acc_sh.at[didx], add=True)
        return carry

    lax.fori_loop(0, NCH, step, 0)
    plsc.subcore_barrier()
    pltpu.sync_copy(acc_sh.at[pl.ds(r0, RPW)], acc_out.at[c, pl.ds(r0, RPW)])


def _dense_body(acc_ref, deg_ref, w0_ref, b0_ref, w1_ref, g_ref):
    a = acc_ref[0] + acc_ref[1]
    d = deg_ref[0] + deg_ref[1]
    dinv = 1.0 / jnp.maximum(d, 1.0)
    m = a * dinv[:, None]
    h = jnp.dot(m, w0_ref[...], preferred_element_type=jnp.float32)
    h = jnp.maximum(h + b0_ref[...], 0.0)
    g_ref[...] = jnp.dot(h, w1_ref[...], preferred_element_type=jnp.float32)


def _final_body(acc_ref, deg_ref, b1_ref, out_ref):
    a = acc_ref[0] + acc_ref[1]
    d = deg_ref[0] + deg_ref[1]
    dinv = 1.0 / jnp.maximum(d, 1.0)
    out_ref[...] = a * dinv[:, None] + b1_ref[...]


@jax.jit
def kernel(x, edge_index, W0, b0, W1, b1):
    npad = EP - E
    src = jnp.concatenate(
        [edge_index[0], jnp.zeros((npad,), jnp.int32)])
    pad_dst = N + (jnp.arange(npad, dtype=jnp.int32) % (NP - N))
    dst = jnp.concatenate([edge_index[1], pad_dst])

    zrows = jnp.zeros((RPW, D), jnp.float32)
    zvec = jnp.zeros((RPW,), jnp.float32)
    ones = jnp.ones((CHUNK,), jnp.float32)

    acc_w, deg = _agg_wide(x, src, dst, zrows, zvec, ones)

    g = pl.pallas_call(
        _dense_body,
        out_shape=jax.ShapeDtypeStruct((NP, C), jnp.float32),
    )(acc_w, deg, W0, b0.reshape(1, D), W1)

    zrows_n = jnp.zeros((RPW, C), jnp.float32)
    acc_n = _agg_narrow(g, src, dst, zrows_n)

    out = pl.pallas_call(
        _final_body,
        out_shape=jax.ShapeDtypeStruct((NP, C), jnp.float32),
    )(acc_n, deg, b1.reshape(1, C))
    return out[:N]
